# baseline (speedup 1.0000x reference)
import functools
import math

import jax
import jax.numpy as jnp
import numpy as np
from jax import lax
from jax.experimental import pallas as pl
from jax.experimental.pallas import tpu as pltpu
from jax.experimental.pallas import tpu_sc as plsc

_CQ = 32
_KP = 128
_SLOTS = 16


_PROWS = _CQ * _SLOTS * 16 // 128


def _sc_m_kernel(ktp_hbm, qc_hbm, sidx_hbm, part_hbm, ib0, ib1, rows0, rows1,
                 q0, q1, part0, part1, sem0, sem1, psem0, psem1, isem0, isem1,
                 *, U: int, D: int, n_chunks: int, nc: int):
    wid = lax.axis_index("s") * nc + lax.axis_index("c")
    n_rows = U * _CQ
    nt = D // 16
    ib = (ib0, ib1)
    rows = (rows0, rows1)
    qb = (q0, q1)
    parts = (part0, part1)
    sems = (sem0, sem1)
    psems = (psem0, psem1)
    isems = (isem0, isem1)
    zeros16 = jnp.zeros((16,), jnp.float32)

    def issue(c, b):
        for j in range(U):
            pltpu.async_copy(ktp_hbm.at[wid].at[ib[b].at[j]],
                             rows[b].at[pl.ds(j * _CQ, _CQ)], sems[b])
        pltpu.async_copy(qc_hbm.at[wid, c], qb[b], sems[b])

    def drain(b):
        pltpu.make_async_copy(ktp_hbm.at[wid, pl.ds(0, n_rows)],
                              rows[b], sems[b]).wait()
        pltpu.make_async_copy(qc_hbm.at[wid, 0], qb[b], sems[b]).wait()

    def drain_part(b):
        pltpu.make_async_copy(
            parts[b], part_hbm.at[wid, pl.ds(0, _PROWS)], psems[b]).wait()

    def drain_idx(b):
        pltpu.make_async_copy(sidx_hbm.at[0], ib[b], isems[b]).wait()

    pltpu.sync_copy(sidx_hbm.at[0], ib[0])
    issue(0, 0)
    pltpu.async_copy(sidx_hbm.at[jnp.minimum(1, n_chunks - 1)], ib[1],
                     isems[1])

    def pair_body(cc, carry):
        for b in range(2):
            c = cc * 2 + b
            cn = jnp.minimum(c + 1, n_chunks - 1)
            cnn = jnp.minimum(c + 2, n_chunks - 1)
            drain_idx(1 - b)
            issue(cn, 1 - b)
            drain(b)
            pltpu.async_copy(sidx_hbm.at[cnn], ib[b], isems[b])

            @pl.when(cc > 0)
            def _():
                drain_part(b)

            def q_body(l, carry2):
                qv = [qb[b][l, pl.ds(t * 16, 16)] for t in range(nt)]
                for s in range(U):
                    r = l * U + s
                    acc = qv[0] * rows[b][r, pl.ds(0, 16)]
                    for t in range(1, nt):
                        acc = acc + qv[t] * rows[b][r, pl.ds(t * 16, 16)]
                    parts[b][2 * l + s // 8, pl.ds(16 * (s % 8), 16)] = acc
                for s in range(U, _SLOTS):
                    parts[b][2 * l + s // 8, pl.ds(16 * (s % 8), 16)] = zeros16
                return carry2

            lax.fori_loop(0, _CQ, q_body, 0)
            pltpu.async_copy(
                parts[b], part_hbm.at[wid, pl.ds(c * _PROWS, _PROWS)],
                psems[b])
        return carry

    lax.fori_loop(0, n_chunks // 2, pair_body, 0)
    drain(0)
    drain_idx((n_chunks - 1) % 2)
    drain_part(0)
    drain_part(1)


def _sc_partials(ktp, qc, sidx, BH: int, L_Q: int, U: int, D: int):
    info = plsc.get_sparse_core_info()
    nc = info.num_cores
    n_chunks = L_Q // _CQ
    mesh = plsc.VectorSubcoreMesh(core_axis_name="c", subcore_axis_name="s")
    kfn = functools.partial(
        pl.kernel,
        mesh=mesh,
        out_type=jax.ShapeDtypeStruct((BH, L_Q * _SLOTS * 16 // 128, 128),
                                      jnp.float32),
        scratch_types=[
            pltpu.VMEM((U, _CQ), jnp.int32),
            pltpu.VMEM((U, _CQ), jnp.int32),
            pltpu.VMEM((U * _CQ, _KP), jnp.float32),
            pltpu.VMEM((U * _CQ, _KP), jnp.float32),
            pltpu.VMEM((_CQ, D), jnp.float32),
            pltpu.VMEM((_CQ, D), jnp.float32),
            pltpu.VMEM((_PROWS, 128), jnp.float32),
            pltpu.VMEM((_PROWS, 128), jnp.float32),
            pltpu.SemaphoreType.DMA,
            pltpu.SemaphoreType.DMA,
            pltpu.SemaphoreType.DMA,
            pltpu.SemaphoreType.DMA,
            pltpu.SemaphoreType.DMA,
            pltpu.SemaphoreType.DMA,
        ],
    )(functools.partial(_sc_m_kernel, U=U, D=D, n_chunks=n_chunks, nc=nc))
    return kfn(ktp, qc, sidx)


def _tc_attn_kernel(q_ref, k_ref, v_ref, p_ref, sidx_ref, out_ref, attn_ref,
                    *, L_K: int, u: int, nc: int, hp: int, D: int,
                    scale: float):
    L_Q = q_ref.shape[0]
    iota = lax.broadcasted_iota(jnp.int32, (L_Q, 1), 0)
    kiota = lax.broadcasted_iota(jnp.int32, (1, L_K), 1)
    siota = lax.broadcasted_iota(jnp.int32, (1, sidx_ref.shape[1]), 1)

    for h in range(hp):
        lanes = slice(h * D, (h + 1) * D)
        k = k_ref[:, lanes]

        p = p_ref[h]
        dots = [jnp.sum(p[:, s * 16:(s + 1) * 16], axis=1, keepdims=True)
                for s in range(u)]
        mx = functools.reduce(jnp.maximum, dots)
        sm = functools.reduce(lambda a, b: a + b, dots)
        m = mx - sm * (1.0 / L_K)

        cands = []
        cur = m
        for _ in range(nc):
            mval = jnp.max(cur)
            j = jnp.min(jnp.where(cur == mval, iota, jnp.int32(2 ** 30)))
            cands.append(j)
            cur = jnp.where(iota == j, -jnp.inf, cur)

        qc = jnp.concatenate([q_ref[pl.ds(j, 1), lanes] for j in cands],
                             axis=0)
        sc = lax.dot_general(qc, k, (((1,), (1,)), ((), ())),
                             preferred_element_type=jnp.float32)
        mex = []
        for ci, j in enumerate(cands):
            srow = sc[ci:ci + 1, :]
            irow = sidx_ref[pl.ds(j, 1), :]
            cntrow = jnp.zeros((1, L_K), jnp.float32)
            for s in range(u):
                vs = jnp.sum(jnp.where(siota == s, irow, 0))
                cntrow = cntrow + (kiota == vs).astype(jnp.float32)
            mxc = jnp.max(jnp.where(cntrow > 0.0, srow, -jnp.inf))
            smc = jnp.sum(srow * cntrow)
            mex.append(jnp.reshape(mxc - smc * (1.0 / L_K), (1, 1)))
        mexact = jnp.concatenate(mex, axis=1)
        ovec = jnp.concatenate(
            [jnp.reshape(j, (1, 1)) for j in cands], axis=1)

        idxs = []
        for _ in range(u):
            mval = jnp.max(mexact)
            j = jnp.min(jnp.where(mexact == mval, ovec, jnp.int32(2 ** 30)))
            idxs.append(j)
            mexact = jnp.where(ovec == j, -jnp.inf, mexact)

        q_rows = [q_ref[pl.ds(j, 1), lanes] for j in idxs]
        qr = jnp.concatenate(q_rows, axis=0)
        scores = lax.dot_general(qr, k, (((1,), (1,)), ((), ())),
                                 preferred_element_type=jnp.float32)
        scores = scores * scale
        smax = jnp.max(scores, axis=1, keepdims=True)
        e = jnp.exp(scores - smax)
        attn = e / jnp.sum(e, axis=1, keepdims=True)
        attn_ref[h] = attn

        v = v_ref[:, lanes]
        upd = lax.dot_general(attn, v, (((1,), (0,)), ((), ())),
                              preferred_element_type=jnp.float32)

        vmean = jnp.sum(v, axis=0, keepdims=True) * (1.0 / L_K)
        out_ref[:, lanes] = jnp.broadcast_to(vmean, (L_Q, D))
        for s_i, j in enumerate(idxs):
            out_ref[pl.ds(j, 1), lanes] = upd[s_i:s_i + 1, :]


def kernel(queries, keys, values):
    B, L_Q, H, D = queries.shape
    _, L_K, _, _ = keys.shape
    factor = 1
    U_part = factor * int(np.ceil(np.log(L_K)))
    u = factor * int(np.ceil(np.log(L_Q)))
    U_part = min(U_part, L_K)
    u = min(u, L_Q)
    scale = 1.0 / math.sqrt(D)
    BH = B * H
    hp = 2

    sidx0 = jax.random.randint(jax.random.key(42), (L_Q, U_part), 0, L_K)
    sidx0 = sidx0.astype(jnp.int32)
    sidx = sidx0.reshape(L_Q // _CQ, U_part, _CQ)
    sidx_pad = jnp.pad(sidx0, ((0, 0), (0, 16 - U_part)))
    nc = min(32, L_Q)

    kt = jnp.transpose(keys, (0, 2, 1, 3)).reshape(BH, L_K, D)
    ktp = jnp.pad(kt, ((0, 0), (0, 0), (0, _KP - D)))
    qc = jnp.transpose(queries, (0, 2, 1, 3)).reshape(BH, L_Q // _CQ, _CQ, D)
    part = _sc_partials(ktp, qc, sidx, BH, L_Q, U_part, D)

    qf = queries.reshape(B, L_Q, H * D)
    kf = keys.reshape(B, L_K, H * D)
    vf = values.reshape(B, L_K, H * D)
    p4 = part.reshape(BH // hp, hp, L_Q, _SLOTS * 16)

    n_hb = H // hp
    grid = (B * n_hb,)
    bh_map = lambda i: (i // n_hb, 0, i % n_hb)

    out, attn = pl.pallas_call(
        functools.partial(_tc_attn_kernel, L_K=L_K, u=u, nc=nc, hp=hp, D=D,
                          scale=scale),
        grid=grid,
        in_specs=[
            pl.BlockSpec((None, L_Q, hp * D), bh_map),
            pl.BlockSpec((None, L_K, hp * D), bh_map),
            pl.BlockSpec((None, L_K, hp * D), bh_map),
            pl.BlockSpec((None, hp, L_Q, _SLOTS * 16), lambda i: (i, 0, 0, 0)),
            pl.BlockSpec((L_Q, 16), lambda i: (0, 0)),
        ],
        out_specs=[
            pl.BlockSpec((None, L_Q, hp * D), bh_map),
            pl.BlockSpec((None, hp, u, L_K),
                         lambda i: (i // n_hb, i % n_hb, 0, 0)),
        ],
        out_shape=[
            jax.ShapeDtypeStruct((B, L_Q, H * D), jnp.float32),
            jax.ShapeDtypeStruct((B, H, u, L_K), jnp.float32),
        ],
        compiler_params=pltpu.CompilerParams(
            dimension_semantics=("arbitrary",),
        ),
    )(qf, kf, vf, p4, sidx_pad)
    return (out.reshape(B, L_Q, H, D), attn)


# speedup vs baseline: 1.1964x; 1.1964x over previous
import functools
import math

import jax
import jax.numpy as jnp
import numpy as np
from jax import lax
from jax.experimental import pallas as pl
from jax.experimental.pallas import tpu as pltpu


@functools.lru_cache(maxsize=None)
def _sample_counts(L_Q: int, L_K: int, U_part: int):
    with jax.ensure_compile_time_eval():
        idx = jax.random.randint(jax.random.key(42), (L_Q, U_part), 0, L_K)
    idx_np = np.asarray(idx, dtype=np.int64)
    cnt = np.zeros((L_Q, L_K), dtype=np.int8)
    np.add.at(cnt, (np.arange(L_Q)[:, None], idx_np), 1)
    return cnt, idx_np.astype(np.int32)


def _prob_attn_kernel(q_ref, k_ref, v_ref, cnt_ref, sidx_ref, out_ref,
                      attn_ref, *, L_K: int, u: int, nc: int, bq: int,
                      hp: int, D: int, scale: float):
    L_Q = q_ref.shape[0]
    iota = lax.broadcasted_iota(jnp.int32, (L_Q, 1), 0)
    kiota = lax.broadcasted_iota(jnp.int32, (1, L_K), 1)
    siota = lax.broadcasted_iota(jnp.int32, (1, sidx_ref.shape[1]), 1)

    for h in range(hp):
        lanes = slice(h * D, (h + 1) * D)
        q = q_ref[:, lanes]
        k = k_ref[:, lanes]
        kb = k.astype(jnp.bfloat16)

        m_cols = []
        for i in range(L_Q // bq):
            qb = q[i * bq:(i + 1) * bq, :].astype(jnp.bfloat16)
            s = lax.dot_general(qb, kb, (((1,), (1,)), ((), ())),
                                preferred_element_type=jnp.float32)
            c = cnt_ref[i * bq:(i + 1) * bq, :].astype(jnp.float32)
            rmax = jnp.max(jnp.where(c > 0.0, s, -jnp.inf), axis=1,
                           keepdims=True)
            rsum = jnp.sum(s * c, axis=1, keepdims=True)
            m_cols.append(rmax - rsum * (1.0 / L_K))
        m = jnp.concatenate(m_cols, axis=0)

        cands = []
        cur = m
        for _ in range(nc):
            mval = jnp.max(cur)
            j = jnp.min(jnp.where(cur == mval, iota, jnp.int32(2 ** 30)))
            cands.append(j)
            cur = jnp.where(iota == j, -jnp.inf, cur)

        qc = jnp.concatenate([q_ref[pl.ds(j, 1), lanes] for j in cands],
                             axis=0)
        sc = lax.dot_general(qc, k, (((1,), (1,)), ((), ())),
                             preferred_element_type=jnp.float32)
        mex = []
        for ci, j in enumerate(cands):
            srow = sc[ci:ci + 1, :]
            irow = sidx_ref[pl.ds(j, 1), :]
            cntrow = jnp.zeros((1, L_K), jnp.float32)
            for s_i in range(u):
                vs = jnp.sum(jnp.where(siota == s_i, irow, 0))
                cntrow = cntrow + (kiota == vs).astype(jnp.float32)
            mxc = jnp.max(jnp.where(cntrow > 0.0, srow, -jnp.inf))
            smc = jnp.sum(srow * cntrow)
            mex.append(jnp.reshape(mxc - smc * (1.0 / L_K), (1, 1)))
        mexact = jnp.concatenate(mex, axis=1)
        ovec = jnp.concatenate(
            [jnp.reshape(j, (1, 1)) for j in cands], axis=1)

        idxs = []
        for _ in range(u):
            mval = jnp.max(mexact)
            j = jnp.min(jnp.where(mexact == mval, ovec, jnp.int32(2 ** 30)))
            idxs.append(j)
            mexact = jnp.where(ovec == j, -jnp.inf, mexact)

        q_rows = [q_ref[pl.ds(j, 1), lanes] for j in idxs]
        qr = jnp.concatenate(q_rows, axis=0)
        scores = lax.dot_general(qr, k, (((1,), (1,)), ((), ())),
                                 preferred_element_type=jnp.float32)
        scores = scores * scale
        smax = jnp.max(scores, axis=1, keepdims=True)
        e = jnp.exp(scores - smax)
        attn = e / jnp.sum(e, axis=1, keepdims=True)
        attn_ref[h] = attn

        v = v_ref[:, lanes]
        upd = lax.dot_general(attn, v, (((1,), (0,)), ((), ())),
                              preferred_element_type=jnp.float32)

        vmean = jnp.sum(v, axis=0, keepdims=True) * (1.0 / L_K)
        out_ref[:, lanes] = jnp.broadcast_to(vmean, (L_Q, D))
        for s_i, j in enumerate(idxs):
            out_ref[pl.ds(j, 1), lanes] = upd[s_i:s_i + 1, :]


def kernel(queries, keys, values):
    B, L_Q, H, D = queries.shape
    _, L_K, _, _ = keys.shape
    factor = 1
    U_part = factor * int(np.ceil(np.log(L_K)))
    u = factor * int(np.ceil(np.log(L_Q)))
    U_part = min(U_part, L_K)
    u = min(u, L_Q)
    scale = 1.0 / math.sqrt(D)
    bq = min(256, L_Q)
    nc = min(32, L_Q)
    hp = 2 if (D == 64 and H % 2 == 0) else 1

    cnt_np, sidx_np = _sample_counts(L_Q, L_K, U_part)
    cnt = jnp.asarray(cnt_np)
    sidx_pad = jnp.asarray(np.pad(sidx_np, ((0, 0), (0, 16 - U_part))))

    qf = queries.reshape(B, L_Q, H * D)
    kf = keys.reshape(B, L_K, H * D)
    vf = values.reshape(B, L_K, H * D)

    n_hb = H // hp
    grid = (B * n_hb,)
    bh_map = lambda i: (i // n_hb, 0, i % n_hb)

    out, attn = pl.pallas_call(
        functools.partial(_prob_attn_kernel, L_K=L_K, u=u, nc=nc, bq=bq,
                          hp=hp, D=D, scale=scale),
        grid=grid,
        in_specs=[
            pl.BlockSpec((None, L_Q, hp * D), bh_map),
            pl.BlockSpec((None, L_K, hp * D), bh_map),
            pl.BlockSpec((None, L_K, hp * D), bh_map),
            pl.BlockSpec((L_Q, L_K), lambda i: (0, 0)),
            pl.BlockSpec((L_Q, 16), lambda i: (0, 0)),
        ],
        out_specs=[
            pl.BlockSpec((None, L_Q, hp * D), bh_map),
            pl.BlockSpec((None, hp, u, L_K),
                         lambda i: (i // n_hb, i % n_hb, 0, 0)),
        ],
        out_shape=[
            jax.ShapeDtypeStruct((B, L_Q, H * D), jnp.float32),
            jax.ShapeDtypeStruct((B, H, u, L_K), jnp.float32),
        ],
        compiler_params=pltpu.CompilerParams(
            dimension_semantics=("arbitrary",),
        ),
    )(qf, kf, vf, cnt, sidx_pad)
    return (out.reshape(B, L_Q, H, D), attn)


# speedup vs baseline: 2.5299x; 2.1147x over previous
import functools
import math

import jax
import jax.numpy as jnp
import numpy as np
from jax.experimental import pallas as pl
from jax.experimental.pallas import tpu as pltpu


@functools.lru_cache(maxsize=None)
def _sample_counts(L_Q: int, L_K: int, U_part: int):
    with jax.ensure_compile_time_eval():
        idx = jax.random.randint(jax.random.key(42), (L_Q, U_part), 0, L_K)
    idx_np = np.asarray(idx, dtype=np.int64)
    cnt = np.zeros((L_Q, L_K), dtype=np.int8)
    np.add.at(cnt, (np.arange(L_Q)[:, None], idx_np), 1)
    return cnt


def _prob_attn_kernel(q_ref, k_ref, v_ref, cnt_ref, out_ref, attn_ref, *,
                      L_K: int, u: int, bq: int, hp: int, D: int,
                      scale: float):
    L_Q = q_ref.shape[0]
    iota = jax.lax.broadcasted_iota(jnp.int32, (L_Q, 1), 0)
    for h in range(hp):
        lanes = slice(h * D, (h + 1) * D)
        q = q_ref[:, lanes]
        k = k_ref[:, lanes]

        m_cols = []
        for i in range(L_Q // bq):
            qb = q[i * bq:(i + 1) * bq, :]
            s = jax.lax.dot_general(qb, k, (((1,), (1,)), ((), ())),
                                    preferred_element_type=jnp.float32)
            c = cnt_ref[i * bq:(i + 1) * bq, :].astype(jnp.float32)
            rmax = jnp.max(jnp.where(c > 0.0, s, -jnp.inf), axis=1,
                           keepdims=True)
            rsum = jnp.sum(s * c, axis=1, keepdims=True)
            m_cols.append(rmax - rsum * (1.0 / L_K))
        m = jnp.concatenate(m_cols, axis=0)

        idxs = []
        cur = m
        for _ in range(u):
            mval = jnp.max(cur)
            j = jnp.min(jnp.where(cur == mval, iota, jnp.int32(2 ** 30)))
            idxs.append(j)
            cur = jnp.where(iota == j, -jnp.inf, cur)

        q_rows = [q_ref[pl.ds(j, 1), lanes] for j in idxs]
        qr = jnp.concatenate(q_rows, axis=0)
        scores = jax.lax.dot_general(qr, k, (((1,), (1,)), ((), ())),
                                     preferred_element_type=jnp.float32)
        scores = scores * scale
        smax = jnp.max(scores, axis=1, keepdims=True)
        e = jnp.exp(scores - smax)
        attn = e / jnp.sum(e, axis=1, keepdims=True)
        attn_ref[h] = attn

        v = v_ref[:, lanes]
        upd = jax.lax.dot_general(attn, v, (((1,), (0,)), ((), ())),
                                  preferred_element_type=jnp.float32)

        vmean = jnp.sum(v, axis=0, keepdims=True) * (1.0 / L_K)
        out_ref[:, lanes] = jnp.broadcast_to(vmean, (L_Q, D))
        for s_i, j in enumerate(idxs):
            out_ref[pl.ds(j, 1), lanes] = upd[s_i:s_i + 1, :]


def kernel(queries, keys, values):
    B, L_Q, H, D = queries.shape
    _, L_K, _, _ = keys.shape
    factor = 1
    U_part = factor * int(np.ceil(np.log(L_K)))
    u = factor * int(np.ceil(np.log(L_Q)))
    U_part = min(U_part, L_K)
    u = min(u, L_Q)
    scale = 1.0 / math.sqrt(D)
    bq = min(256, L_Q)
    hp = 2 if (D == 64 and H % 2 == 0) else 1

    cnt = jnp.asarray(_sample_counts(L_Q, L_K, U_part))
    qf = queries.reshape(B, L_Q, H * D)
    kf = keys.reshape(B, L_K, H * D)
    vf = values.reshape(B, L_K, H * D)

    n_hb = H // hp
    grid = (B * n_hb,)
    bh_map = lambda i: (i // n_hb, 0, i % n_hb)

    out, attn = pl.pallas_call(
        functools.partial(_prob_attn_kernel, L_K=L_K, u=u, bq=bq, hp=hp, D=D,
                          scale=scale),
        grid=grid,
        in_specs=[
            pl.BlockSpec((None, L_Q, hp * D), bh_map),
            pl.BlockSpec((None, L_K, hp * D), bh_map),
            pl.BlockSpec((None, L_K, hp * D), bh_map),
            pl.BlockSpec((L_Q, L_K), lambda i: (0, 0)),
        ],
        out_specs=[
            pl.BlockSpec((None, L_Q, hp * D), bh_map),
            pl.BlockSpec((None, hp, u, L_K),
                         lambda i: (i // n_hb, i % n_hb, 0, 0)),
        ],
        out_shape=[
            jax.ShapeDtypeStruct((B, L_Q, H * D), jnp.float32),
            jax.ShapeDtypeStruct((B, H, u, L_K), jnp.float32),
        ],
        compiler_params=pltpu.CompilerParams(
            dimension_semantics=("arbitrary",),
        ),
    )(qf, kf, vf, cnt)
    return (out.reshape(B, L_Q, H, D), attn)
